# baseline (speedup 1.0000x reference)
import functools

import jax
import jax.numpy as jnp
from jax import lax
from jax.experimental import pallas as pl
from jax.experimental.pallas import tpu as pltpu
from jax.experimental.pallas import tpu_sc as plsc

EMB = 256
N_TOK = 8192
N_CODES = 8192
TB = 512
KB = 2048
GB = 64


ROWS_PER_BLK = KB // 128
N_ROWS = N_CODES // 128


def _argmin_body(segs, zz_ref, ee_ref, z_ref, e_ref, idx_ref,
                 rmin_ref, rarg_ref):
    j = pl.program_id(1)
    cross = lax.dot_general(z_ref[...], e_ref[...], (((1,), (1,)), ((), ())),
                            preferred_element_type=jnp.float32)
    d = zz_ref[0] + ee_ref[0] - 2.0 * cross
    mins, args = [], []
    for g in range(ROWS_PER_BLK):
        blk = d[:, g * 128:(g + 1) * 128]
        m = jnp.min(blk, axis=1, keepdims=True)
        lanes = lax.broadcasted_iota(jnp.int32, blk.shape, 1)
        a = jnp.min(jnp.where(blk == m, lanes, 128), axis=1, keepdims=True)
        mins.append(m)
        args.append(a + (j * KB + g * 128))
    rmin_ref[j] = jnp.concatenate(mins, axis=1)
    rarg_ref[j] = jnp.concatenate(args, axis=1)

    @pl.when(j == pl.num_programs(1) - 1)
    def _():
        R = jnp.concatenate([rmin_ref[k] for k in range(N_CODES // KB)], axis=1)
        A = jnp.concatenate([rarg_ref[k] for k in range(N_CODES // KB)], axis=1)
        allrows = lax.broadcasted_iota(jnp.int32, (TB, N_ROWS), 1)
        acc_v = acc_a = None
        for (s, e, w) in segs:
            sub = R[:, s:e]
            m = jnp.min(sub, axis=1, keepdims=True)
            rows = lax.broadcasted_iota(jnp.int32, sub.shape, 1) + s
            r1 = jnp.min(jnp.where(sub == m, rows, N_ROWS), axis=1,
                         keepdims=True)
            a = jnp.sum(jnp.where(allrows == r1, A, 0), axis=1, keepdims=True)
            if acc_v is None:
                acc_v, acc_a = m, a
            else:
                rounded = acc_v.astype(jnp.bfloat16).astype(jnp.float32)
                cmp = jnp.where(acc_a >= s * 128 - w, acc_v, rounded)
                take = m < cmp
                acc_v = jnp.where(take, m, acc_v)
                acc_a = jnp.where(take, a, acc_a)
        idx_ref[0] = acc_a


def _argmin_call(z_flat, emb, zz, ee, segs):
    nt, nk = N_TOK // TB, N_CODES // KB
    out = pl.pallas_call(
        functools.partial(_argmin_body, segs),
        grid=(nt, nk),
        in_specs=[
            pl.BlockSpec((1, TB, 1), lambda i, j: (i, 0, 0)),
            pl.BlockSpec((1, 1, KB), lambda i, j: (j, 0, 0)),
            pl.BlockSpec((TB, EMB), lambda i, j: (i, 0)),
            pl.BlockSpec((KB, EMB), lambda i, j: (j, 0)),
        ],
        out_specs=pl.BlockSpec((1, TB, 1), lambda i, j: (i, 0, 0)),
        out_shape=jax.ShapeDtypeStruct((nt, TB, 1), jnp.int32),
        scratch_shapes=[
            pltpu.VMEM((nk, TB, ROWS_PER_BLK), jnp.float32),
            pltpu.VMEM((nk, TB, ROWS_PER_BLK), jnp.int32),
        ],
    )(zz.reshape(nt, TB, 1), ee.reshape(nk, 1, KB), z_flat, emb)
    return out.reshape(N_TOK)


SEGS_STRUCT = ((0, 32, 0), (32, 64, 0))
SEGS_STYLE = ((0, 13, 0), (13, 25, 0), (25, 37, 0), (37, 49, 0), (49, 64, 0))


def _gather_sc(tab_s, tab_t, idx_s, idx_t):
    info = plsc.get_sparse_core_info()
    nc, ns = info.num_cores, info.num_subcores
    nw = nc * ns
    bpw = N_TOK // nw
    mesh = plsc.VectorSubcoreMesh(core_axis_name="c", subcore_axis_name="s",
                                  num_cores=nc, num_subcores=ns)

    @functools.partial(
        pl.kernel, mesh=mesh,
        out_type=[jax.ShapeDtypeStruct((N_TOK, EMB), jnp.float32),
                  jax.ShapeDtypeStruct((N_TOK, EMB), jnp.float32)],
        scratch_types=[pltpu.VMEM((bpw,), jnp.int32),
                       pltpu.VMEM((bpw, EMB), jnp.float32),
                       pltpu.SemaphoreType.DMA],
    )
    def k(ts_hbm, tt_hbm, is_hbm, it_hbm, os_hbm, ot_hbm, idx_v, rows_v, sem):
        wid = lax.axis_index("s") * nc + lax.axis_index("c")
        base = wid * bpw
        pltpu.sync_copy(is_hbm.at[pl.ds(base, bpw)], idx_v)
        pltpu.async_copy(ts_hbm.at[idx_v], rows_v, sem).wait()
        pltpu.sync_copy(rows_v, os_hbm.at[pl.ds(base, bpw)])
        pltpu.sync_copy(it_hbm.at[pl.ds(base, bpw)], idx_v)
        pltpu.async_copy(tt_hbm.at[idx_v], rows_v, sem).wait()
        pltpu.sync_copy(rows_v, ot_hbm.at[pl.ds(base, bpw)])

    return k(tab_s, tab_t, idx_s, idx_t)


def _combine_body(s_ref, t_ref, out_ref, loss_ref, acc_ref):
    b = pl.program_id(0)
    s = s_ref[...]
    t = t_ref[...]
    avg = ((s + t) * 0.5).reshape(GB * 16, EMB)
    out_ref[0] = avg.T
    sn = s / jnp.maximum(jnp.sqrt(jnp.sum(s * s, axis=1, keepdims=True)), 1e-12)
    tn = t / jnp.maximum(jnp.sqrt(jnp.sum(t * t, axis=1, keepdims=True)), 1e-12)
    dot = jnp.sum(sn * tn, axis=1)
    part = jnp.sum(dot * dot)
    acc = jnp.where(b == 0, 0.0, acc_ref[0, 0]) + part
    acc_ref[0, 0] = acc

    @pl.when(b == pl.num_programs(0) - 1)
    def _():
        loss_ref[...] = jnp.full((1, 1), acc / float(2 * EMB * 16 * 16),
                                 jnp.float32)


def _combine_call(zq_s3, zq_t3):
    ngrp = N_TOK // 16
    nblk = ngrp // GB
    toks = GB * 16
    zq, loss = pl.pallas_call(
        _combine_body,
        grid=(nblk,),
        in_specs=[
            pl.BlockSpec((GB, 16, EMB), lambda b: (b, 0, 0)),
            pl.BlockSpec((GB, 16, EMB), lambda b: (b, 0, 0)),
        ],
        out_specs=[
            pl.BlockSpec((1, EMB, toks), lambda b: (b // 4, 0, b % 4)),
            pl.BlockSpec((1, 1), lambda b: (0, 0)),
        ],
        out_shape=[
            jax.ShapeDtypeStruct((2, EMB, N_TOK // 2), jnp.float32),
            jax.ShapeDtypeStruct((1, 1), jnp.float32),
        ],
        scratch_shapes=[pltpu.SMEM((1, 1), jnp.float32)],
    )(zq_s3, zq_t3)
    return zq, loss


def kernel(z, structure_embedding, style_embedding):
    zp = jnp.transpose(z, (0, 2, 3, 4, 1))
    z_flat = zp.reshape(-1, EMB)
    zz = jnp.sum(z_flat ** 2, axis=1)
    ee_s = jnp.sum(structure_embedding ** 2, axis=1)
    ee_t = jnp.sum(style_embedding ** 2, axis=1)
    idx_s = _argmin_call(z_flat, structure_embedding, zz, ee_s, SEGS_STRUCT)
    idx_t = _argmin_call(z_flat, style_embedding, zz, ee_t, SEGS_STYLE)
    tab_s = structure_embedding.astype(jnp.bfloat16).astype(jnp.float32)
    tab_t = style_embedding.astype(jnp.bfloat16).astype(jnp.float32)
    zq_s, zq_t = _gather_sc(tab_s, tab_t, idx_s, idx_t)
    zq, loss = _combine_call(zq_s.reshape(N_TOK // 16, 16, EMB),
                             zq_t.reshape(N_TOK // 16, 16, EMB))
    z_q = zq.reshape(2, EMB, 16, 16, 16)
    return (z_q, loss.reshape(()), idx_s, idx_t)


# speedup vs baseline: 1.0164x; 1.0164x over previous
import functools

import jax
import jax.numpy as jnp
from jax import lax
from jax.experimental import pallas as pl
from jax.experimental.pallas import tpu as pltpu
from jax.experimental.pallas import tpu_sc as plsc

EMB = 256
N_TOK = 8192
N_CODES = 8192
TB = 512
KB = 2048
GB = 64


ROWS_PER_BLK = KB // 128
N_ROWS = N_CODES // 128


def _argmin_body(segs, zz_ref, ee_ref, z_ref, e_ref, idx_ref):
    cross = lax.dot_general(z_ref[...], e_ref[...], (((1,), (1,)), ((), ())),
                            preferred_element_type=jnp.float32)
    d = zz_ref[0] + ee_ref[0] - 2.0 * cross
    mins, args = [], []
    for g in range(N_ROWS):
        blk = d[:, g * 128:(g + 1) * 128]
        m = jnp.min(blk, axis=1, keepdims=True)
        lanes = lax.broadcasted_iota(jnp.int32, blk.shape, 1)
        a = jnp.min(jnp.where(blk == m, lanes, 128), axis=1, keepdims=True)
        mins.append(m)
        args.append(a + g * 128)
    R = jnp.concatenate(mins, axis=1)
    A = jnp.concatenate(args, axis=1)
    allrows = lax.broadcasted_iota(jnp.int32, (TB, N_ROWS), 1)
    acc_v = acc_a = None
    for (s, e, w) in segs:
        sub = R[:, s:e]
        m = jnp.min(sub, axis=1, keepdims=True)
        rows = lax.broadcasted_iota(jnp.int32, sub.shape, 1) + s
        r1 = jnp.min(jnp.where(sub == m, rows, N_ROWS), axis=1, keepdims=True)
        a = jnp.sum(jnp.where(allrows == r1, A, 0), axis=1, keepdims=True)
        if acc_v is None:
            acc_v, acc_a = m, a
        else:
            rounded = acc_v.astype(jnp.bfloat16).astype(jnp.float32)
            cmp = jnp.where(acc_a >= s * 128 - w, acc_v, rounded)
            take = m < cmp
            acc_v = jnp.where(take, m, acc_v)
            acc_a = jnp.where(take, a, acc_a)
    idx_ref[0] = acc_a


def _argmin_call(z_flat, emb, zz, ee, segs):
    nt = N_TOK // TB
    out = pl.pallas_call(
        functools.partial(_argmin_body, segs),
        grid=(nt,),
        in_specs=[
            pl.BlockSpec((1, TB, 1), lambda i: (i, 0, 0)),
            pl.BlockSpec((1, 1, N_CODES), lambda i: (0, 0, 0)),
            pl.BlockSpec((TB, EMB), lambda i: (i, 0)),
            pl.BlockSpec((N_CODES, EMB), lambda i: (0, 0)),
        ],
        out_specs=pl.BlockSpec((1, TB, 1), lambda i: (i, 0, 0)),
        out_shape=jax.ShapeDtypeStruct((nt, TB, 1), jnp.int32),
    )(zz.reshape(nt, TB, 1), ee.reshape(1, 1, N_CODES),
      z_flat.astype(jnp.bfloat16), emb.astype(jnp.bfloat16))
    return out.reshape(N_TOK)


SEGS_STRUCT = ((0, 32, 0), (32, 64, 0))
SEGS_STYLE = ((0, 13, 0), (13, 25, 0), (25, 37, 0), (37, 49, 0), (49, 64, 0))


def _gather_sc(tab_s, tab_t, idx_s, idx_t):
    info = plsc.get_sparse_core_info()
    nc, ns = info.num_cores, info.num_subcores
    nw = nc * ns
    bpw = N_TOK // nw
    mesh = plsc.VectorSubcoreMesh(core_axis_name="c", subcore_axis_name="s",
                                  num_cores=nc, num_subcores=ns)

    @functools.partial(
        pl.kernel, mesh=mesh,
        out_type=[jax.ShapeDtypeStruct((N_TOK, EMB), jnp.float32),
                  jax.ShapeDtypeStruct((N_TOK, EMB), jnp.float32)],
        scratch_types=[pltpu.VMEM((bpw,), jnp.int32),
                       pltpu.VMEM((bpw, EMB), jnp.float32),
                       pltpu.SemaphoreType.DMA],
    )
    def k(ts_hbm, tt_hbm, is_hbm, it_hbm, os_hbm, ot_hbm, idx_v, rows_v, sem):
        wid = lax.axis_index("s") * nc + lax.axis_index("c")
        base = wid * bpw
        pltpu.sync_copy(is_hbm.at[pl.ds(base, bpw)], idx_v)
        pltpu.async_copy(ts_hbm.at[idx_v], rows_v, sem).wait()
        pltpu.sync_copy(rows_v, os_hbm.at[pl.ds(base, bpw)])
        pltpu.sync_copy(it_hbm.at[pl.ds(base, bpw)], idx_v)
        pltpu.async_copy(tt_hbm.at[idx_v], rows_v, sem).wait()
        pltpu.sync_copy(rows_v, ot_hbm.at[pl.ds(base, bpw)])

    return k(tab_s, tab_t, idx_s, idx_t)


def _combine_body(s_ref, t_ref, out_ref, loss_ref, acc_ref):
    b = pl.program_id(0)
    s = s_ref[...]
    t = t_ref[...]
    avg = ((s + t) * 0.5).reshape(GB * 16, EMB)
    out_ref[0] = avg.T
    sn = s / jnp.maximum(jnp.sqrt(jnp.sum(s * s, axis=1, keepdims=True)), 1e-12)
    tn = t / jnp.maximum(jnp.sqrt(jnp.sum(t * t, axis=1, keepdims=True)), 1e-12)
    dot = jnp.sum(sn * tn, axis=1)
    part = jnp.sum(dot * dot)
    acc = jnp.where(b == 0, 0.0, acc_ref[0, 0]) + part
    acc_ref[0, 0] = acc

    @pl.when(b == pl.num_programs(0) - 1)
    def _():
        loss_ref[...] = jnp.full((1, 1), acc / float(2 * EMB * 16 * 16),
                                 jnp.float32)


def _combine_call(zq_s3, zq_t3):
    ngrp = N_TOK // 16
    nblk = ngrp // GB
    toks = GB * 16
    zq, loss = pl.pallas_call(
        _combine_body,
        grid=(nblk,),
        in_specs=[
            pl.BlockSpec((GB, 16, EMB), lambda b: (b, 0, 0)),
            pl.BlockSpec((GB, 16, EMB), lambda b: (b, 0, 0)),
        ],
        out_specs=[
            pl.BlockSpec((1, EMB, toks), lambda b: (b // 4, 0, b % 4)),
            pl.BlockSpec((1, 1), lambda b: (0, 0)),
        ],
        out_shape=[
            jax.ShapeDtypeStruct((2, EMB, N_TOK // 2), jnp.float32),
            jax.ShapeDtypeStruct((1, 1), jnp.float32),
        ],
        scratch_shapes=[pltpu.SMEM((1, 1), jnp.float32)],
    )(zq_s3, zq_t3)
    return zq, loss


def kernel(z, structure_embedding, style_embedding):
    zp = jnp.transpose(z, (0, 2, 3, 4, 1))
    z_flat = zp.reshape(-1, EMB)
    zz = jnp.sum(z_flat ** 2, axis=1)
    ee_s = jnp.sum(structure_embedding ** 2, axis=1)
    ee_t = jnp.sum(style_embedding ** 2, axis=1)
    idx_s = _argmin_call(z_flat, structure_embedding, zz, ee_s, SEGS_STRUCT)
    idx_t = _argmin_call(z_flat, style_embedding, zz, ee_t, SEGS_STYLE)
    tab_s = structure_embedding.astype(jnp.bfloat16).astype(jnp.float32)
    tab_t = style_embedding.astype(jnp.bfloat16).astype(jnp.float32)
    zq_s, zq_t = _gather_sc(tab_s, tab_t, idx_s, idx_t)
    zq, loss = _combine_call(zq_s.reshape(N_TOK // 16, 16, EMB),
                             zq_t.reshape(N_TOK // 16, 16, EMB))
    z_q = zq.reshape(2, EMB, 16, 16, 16)
    return (z_q, loss.reshape(()), idx_s, idx_t)
